# baseline (speedup 1.0000x reference)
import functools

import jax
import jax.numpy as jnp
from jax import lax
from jax.experimental import pallas as pl
from jax.experimental.pallas import tpu as pltpu
from jax.experimental.pallas import tpu_sc as plsc

NUM_ITEMS = 1000000
NPAIR = NUM_ITEMS // 2
B = 4096
S = 50
S_PAD = 64
E = 64
NW = 32

ITEMS_PER_TILE = (B * S) // NW
ITEM_CHUNK = 128
N_ITEM_CHUNKS = ITEMS_PER_TILE // ITEM_CHUNK
B_PER_TILE = B // NW
B_GROUP = 4
N_B_GROUPS = B_PER_TILE // B_GROUP
CTX_CHUNK = B_GROUP * S_PAD
POOL_ROWS = 8


def _sc_gather_pool(itab2, ctab2, ipair, cpair, coff):
    mesh = plsc.VectorSubcoreMesh(core_axis_name="c", subcore_axis_name="s")

    @functools.partial(
        pl.kernel,
        mesh=mesh,
        out_type=[
            jax.ShapeDtypeStruct((B * S, 2 * E), jnp.float32),
            jax.ShapeDtypeStruct((B, E), jnp.float32),
        ],
        scratch_types=[
            pltpu.VMEM((ITEM_CHUNK,), jnp.int32),
            pltpu.VMEM((ITEM_CHUNK,), jnp.int32),
            pltpu.VMEM((ITEM_CHUNK,), jnp.int32),
            pltpu.VMEM((ITEM_CHUNK, 2 * E), jnp.float32),
            pltpu.VMEM((ITEM_CHUNK, 2 * E), jnp.float32),
            pltpu.VMEM((ITEM_CHUNK, 2 * E), jnp.float32),
            pltpu.VMEM((CTX_CHUNK,), jnp.int32),
            pltpu.VMEM((CTX_CHUNK,), jnp.int32),
            pltpu.VMEM((CTX_CHUNK + 16,), jnp.int32),
            pltpu.VMEM((CTX_CHUNK + 16,), jnp.int32),
            pltpu.VMEM((CTX_CHUNK, 2 * E), jnp.float32),
            pltpu.VMEM((CTX_CHUNK, 2 * E), jnp.float32),
            pltpu.VMEM((POOL_ROWS, E), jnp.float32),
            pltpu.VMEM((POOL_ROWS, E), jnp.float32),
            pltpu.SemaphoreType.DMA,
            pltpu.SemaphoreType.DMA,
            pltpu.SemaphoreType.DMA,
            pltpu.SemaphoreType.DMA,
            pltpu.SemaphoreType.DMA,
            pltpu.SemaphoreType.DMA,
            pltpu.SemaphoreType.DMA,
            pltpu.SemaphoreType.DMA,
            pltpu.SemaphoreType.DMA,
            pltpu.SemaphoreType.DMA,
        ],
        compiler_params=pltpu.CompilerParams(use_tc_tiling_on_sc=True),
    )
    def k(itab_hbm, ctab_hbm, ipair_hbm, cpair_hbm, coff_hbm,
          item_out, pooled_out,
          iidx0, iidx1, iidx2, ibuf0, ibuf1, ibuf2,
          cidx0, cidx1, coff0, coff1, cbuf0, cbuf1, pool0, pool1,
          sem_ig0, sem_ig1, sem_ig2, sem_iw0, sem_iw1, sem_iw2,
          sem_cg0, sem_cg1, sem_pw0, sem_pw1):
        wid = lax.axis_index("s") * 2 + lax.axis_index("c")
        iidx = (iidx0, iidx1, iidx2)
        ibuf = (ibuf0, ibuf1, ibuf2)
        cidx = (cidx0, cidx1)
        coff_v = (coff0, coff1)
        cbuf = (cbuf0, cbuf1)
        pool = (pool0, pool1)
        sem_ig = (sem_ig0, sem_ig1, sem_ig2)
        sem_iw = (sem_iw0, sem_iw1, sem_iw2)
        sem_cg = (sem_cg0, sem_cg1)
        sem_pw = (sem_pw0, sem_pw1)

        def istart(t, j):
            base = pl.multiple_of(wid * ITEMS_PER_TILE + t * ITEM_CHUNK,
                                  ITEM_CHUNK)
            pltpu.sync_copy(ipair_hbm.at[pl.ds(base, ITEM_CHUNK)], iidx[j])
            pltpu.async_copy(itab_hbm.at[iidx[j]], ibuf[j], sem_ig[j])

        def iwait_g(j):
            pltpu.make_async_copy(itab_hbm.at[iidx[j]], ibuf[j],
                                  sem_ig[j]).wait()

        def iwstart(t, j):
            base = pl.multiple_of(wid * ITEMS_PER_TILE + t * ITEM_CHUNK,
                                  ITEM_CHUNK)
            pltpu.async_copy(ibuf[j], item_out.at[pl.ds(base, ITEM_CHUNK)],
                             sem_iw[j])

        def iwait_w(j):
            pltpu.make_async_copy(
                ibuf[j], item_out.at[pl.ds(0, ITEM_CHUNK)], sem_iw[j]).wait()

        def cstart(g, b):
            cbase = pl.multiple_of(
                wid * (B_PER_TILE * S_PAD) + g * CTX_CHUNK, CTX_CHUNK)
            pltpu.sync_copy(cpair_hbm.at[pl.ds(cbase, CTX_CHUNK)], cidx[b])
            pltpu.sync_copy(coff_hbm.at[pl.ds(cbase, CTX_CHUNK)],
                            coff_v[b].at[pl.ds(0, CTX_CHUNK)])
            pltpu.async_copy(ctab_hbm.at[cidx[b]], cbuf[b], sem_cg[b])

        def cwait_g(b):
            pltpu.make_async_copy(ctab_hbm.at[cidx[b]], cbuf[b],
                                  sem_cg[b]).wait()

        def pwait(ps):
            pltpu.make_async_copy(
                pool[ps], pooled_out.at[pl.ds(0, POOL_ROWS)],
                sem_pw[ps]).wait()

        def accumulate(b, half):
            zero = jnp.zeros((16,), jnp.float32)
            for bi in range(B_GROUP):
                def abody(s, acc, _bi=bi, _b=b):
                    j = _bi * S_PAD + s
                    off = coff_v[_b][pl.ds(j, 16)][0]
                    return tuple(
                        acc[c] + cbuf[_b][j, pl.ds(off + c * 16, 16)]
                        for c in range(4))

                acc = lax.fori_loop(0, S, abody, (zero, zero, zero, zero))
                for c in range(4):
                    pool[half[0]][half[1] * B_GROUP + bi,
                                  pl.ds(c * 16, 16)] = acc[c]

        istart(0, 0)
        cstart(0, 0)
        cstart(1, 1)

        def outer(q, carry):
            for psel in range(2):
                p = 2 * q + psel
                g0 = 4 * q + 2 * psel

                @pl.when(q >= 1)
                def _(_psel=psel):
                    pwait(_psel)

                cwait_g(0)
                accumulate(0, (psel, 0))

                @pl.when(g0 + 2 < N_B_GROUPS)
                def _(_g0=g0):
                    cstart(_g0 + 2, 0)

                for j in range(3):
                    t = 6 * q + 3 * psel + j
                    iwait_g(j)
                    iwstart(t, j)

                    @pl.when(t >= 2)
                    def _(_j=j):
                        iwait_w((_j + 1) % 3)

                    @pl.when(t + 1 < N_ITEM_CHUNKS)
                    def _(_j=j, _t=t):
                        istart(_t + 1, (_j + 1) % 3)

                cwait_g(1)
                accumulate(1, (psel, 1))

                @pl.when(g0 + 3 < N_B_GROUPS)
                def _(_g0=g0):
                    cstart(_g0 + 3, 1)

                obase = pl.multiple_of(wid * B_PER_TILE + p * POOL_ROWS,
                                       POOL_ROWS)
                pltpu.async_copy(pool[psel],
                                 pooled_out.at[pl.ds(obase, POOL_ROWS)],
                                 sem_pw[psel])
            return carry

        lax.fori_loop(0, N_B_GROUPS // 4, outer, 0)

        iwait_w((N_ITEM_CHUNKS - 4) % 3)
        istart(N_ITEM_CHUNKS - 1, (N_ITEM_CHUNKS - 1) % 3)
        iwait_g((N_ITEM_CHUNKS - 2) % 3)
        iwstart(N_ITEM_CHUNKS - 2, (N_ITEM_CHUNKS - 2) % 3)
        iwait_g((N_ITEM_CHUNKS - 1) % 3)
        iwstart(N_ITEM_CHUNKS - 1, (N_ITEM_CHUNKS - 1) % 3)
        iwait_w((N_ITEM_CHUNKS - 3) % 3)
        iwait_w((N_ITEM_CHUNKS - 2) % 3)
        iwait_w((N_ITEM_CHUNKS - 1) % 3)
        pwait(0)
        pwait(1)

    return k(itab2, ctab2, ipair, cpair, coff)


def _tc_score(item2, iparity, cidx, pooled, row0, W1, b1, W2, b2):
    BB = 256

    def body(item_ref, par_ref, cidx_ref, pool_ref, row0_ref, W1_ref, b1_ref,
             W2_ref, b2_ref, out_ref):
        nz = jnp.sum((cidx_ref[...] == 0).astype(jnp.float32), axis=1,
                     keepdims=True)
        pooled_c = pool_ref[...] - nz * row0_ref[...]
        up = jnp.tanh(
            jnp.dot(pooled_c, W1_ref[...],
                    preferred_element_type=jnp.float32) + b1_ref[...])
        ctx = (jnp.dot(up, W2_ref[...], preferred_element_type=jnp.float32)
               + b2_ref[...])
        item3 = item_ref[...].reshape(BB, S, 2 * E)
        ctx128 = jnp.concatenate([ctx, ctx], axis=-1)
        diff = item3 - ctx128[:, None, :]
        sq = diff * diff
        lane = lax.broadcasted_iota(jnp.int32, (BB, S, 2 * E), 2)
        sel = (lane // E) == par_ref[...][:, :, None]
        d2 = jnp.sum(jnp.where(sel, sq, 0.0), axis=-1)
        out_ref[...] = 1.0 - jnp.tanh(jnp.sqrt(d2))

    return pl.pallas_call(
        body,
        grid=(B // BB,),
        in_specs=[
            pl.BlockSpec((BB * S, 2 * E), lambda i: (i, 0)),
            pl.BlockSpec((BB, S), lambda i: (i, 0)),
            pl.BlockSpec((BB, S), lambda i: (i, 0)),
            pl.BlockSpec((BB, E), lambda i: (i, 0)),
            pl.BlockSpec((1, E), lambda i: (0, 0)),
            pl.BlockSpec((E, 2 * E), lambda i: (0, 0)),
            pl.BlockSpec((1, 2 * E), lambda i: (0, 0)),
            pl.BlockSpec((2 * E, E), lambda i: (0, 0)),
            pl.BlockSpec((1, E), lambda i: (0, 0)),
        ],
        out_specs=pl.BlockSpec((BB, S), lambda i: (i, 0)),
        out_shape=jax.ShapeDtypeStruct((B, S), jnp.float32),
    )(item2, iparity, cidx, pooled, row0, W1, b1, W2, b2)


def kernel(item_indices, context_indices, item_table, ctx_table, W1, b1, W2, b2):
    ii = item_indices.astype(jnp.int32)
    ci = context_indices.astype(jnp.int32)
    ipair = (ii >> 1).reshape(-1)
    iparity = ii & 1
    spread = (jnp.arange(B * (S_PAD - S), dtype=jnp.int32) * 7919) % NPAIR
    cpair = jnp.concatenate([ci >> 1, spread.reshape(B, S_PAD - S)], axis=1)
    coff = jnp.pad(ci & 1, ((0, 0), (0, S_PAD - S))) * E
    item2, pooled = _sc_gather_pool(
        item_table.reshape(NPAIR, 2 * E), ctx_table.reshape(NPAIR, 2 * E),
        ipair, cpair.reshape(-1), coff.reshape(-1))
    row0 = lax.slice(ctx_table, (0, 0), (1, E))
    return _tc_score(item2, iparity, ci, pooled, row0,
                     W1, b1.reshape(1, -1), W2, b2.reshape(1, -1))


# speedup vs baseline: 1.0497x; 1.0497x over previous
import functools

import jax
import jax.numpy as jnp
from jax import lax
from jax.experimental import pallas as pl
from jax.experimental.pallas import tpu as pltpu
from jax.experimental.pallas import tpu_sc as plsc

NUM_ITEMS = 1000000
B = 4096
S = 50
S_PAD = 56
E = 64
NW = 32

ITEMS_PER_TILE = (B * S) // NW
ITEM_CHUNK = 320
N_ITEM_CHUNKS = ITEMS_PER_TILE // ITEM_CHUNK
B_PER_TILE = B // NW
B_GROUP = 8
N_B_GROUPS = B_PER_TILE // B_GROUP
CTX_CHUNK = B_GROUP * S_PAD

_MESH = plsc.VectorSubcoreMesh(core_axis_name="c", subcore_axis_name="s")
_SC_PARAMS = pltpu.CompilerParams(use_tc_tiling_on_sc=False)


def _sc_item_gather(itab, iidx_flat):
    @functools.partial(
        pl.kernel,
        mesh=_MESH,
        out_type=jax.ShapeDtypeStruct((B * S, E), jnp.float32),
        scratch_types=[
            pltpu.VMEM((ITEM_CHUNK,), jnp.int32),
            pltpu.VMEM((ITEM_CHUNK,), jnp.int32),
            pltpu.VMEM((ITEM_CHUNK, E), jnp.float32),
            pltpu.VMEM((ITEM_CHUNK, E), jnp.float32),
            pltpu.SemaphoreType.DMA,
            pltpu.SemaphoreType.DMA,
            pltpu.SemaphoreType.DMA,
            pltpu.SemaphoreType.DMA,
        ],
        compiler_params=_SC_PARAMS,
    )
    def k(itab_hbm, iidx_hbm, item_out,
          iidx0, iidx1, ibuf0, ibuf1, sem_g0, sem_g1, sem_w0, sem_w1):
        wid = lax.axis_index("s") * 2 + lax.axis_index("c")
        iidx = (iidx0, iidx1)
        ibuf = (ibuf0, ibuf1)
        sem_g = (sem_g0, sem_g1)
        sem_w = (sem_w0, sem_w1)

        def istart(kk):
            bsel = kk % 2
            base = pl.multiple_of(wid * ITEMS_PER_TILE + kk * ITEM_CHUNK,
                                  ITEM_CHUNK)
            pltpu.sync_copy(iidx_hbm.at[pl.ds(base, ITEM_CHUNK)], iidx[bsel])
            return pltpu.async_copy(itab_hbm.at[iidx[bsel]], ibuf[bsel],
                                    sem_g[bsel])

        def iwrite(kk):
            bsel = kk % 2
            base = pl.multiple_of(wid * ITEMS_PER_TILE + kk * ITEM_CHUNK,
                                  ITEM_CHUNK)
            return pltpu.async_copy(
                ibuf[bsel], item_out.at[pl.ds(base, ITEM_CHUNK)],
                sem_w[bsel])

        gathers = [istart(0)]
        writes = []
        for kk in range(1, N_ITEM_CHUNKS):
            if kk >= 2:
                writes[kk - 2].wait()
            gathers.append(istart(kk))
            gathers[kk - 1].wait()
            writes.append(iwrite(kk - 1))
        gathers[N_ITEM_CHUNKS - 1].wait()
        writes.append(iwrite(N_ITEM_CHUNKS - 1))
        writes[N_ITEM_CHUNKS - 2].wait()
        writes[N_ITEM_CHUNKS - 1].wait()

    return k(itab, iidx_flat)


def _sc_ctx_pool(ctab, cidx_flat):
    @functools.partial(
        pl.kernel,
        mesh=_MESH,
        out_type=jax.ShapeDtypeStruct((B, E), jnp.float32),
        scratch_types=[
            pltpu.VMEM((CTX_CHUNK,), jnp.int32),
            pltpu.VMEM((CTX_CHUNK,), jnp.int32),
            pltpu.VMEM((CTX_CHUNK, E), jnp.float32),
            pltpu.VMEM((CTX_CHUNK, E), jnp.float32),
            pltpu.VMEM((B_GROUP, E), jnp.float32),
            pltpu.VMEM((B_GROUP, E), jnp.float32),
            pltpu.SemaphoreType.DMA,
            pltpu.SemaphoreType.DMA,
            pltpu.SemaphoreType.DMA,
            pltpu.SemaphoreType.DMA,
        ],
        compiler_params=_SC_PARAMS,
    )
    def k(ctab_hbm, cidx_hbm, pooled_out,
          cidx0, cidx1, cbuf0, cbuf1, pool0, pool1,
          sem_g0, sem_g1, sem_p0, sem_p1):
        wid = lax.axis_index("s") * 2 + lax.axis_index("c")
        cidx = (cidx0, cidx1)
        cbuf = (cbuf0, cbuf1)
        pool = (pool0, pool1)
        sem_g = (sem_g0, sem_g1)
        sem_p = (sem_p0, sem_p1)

        def cstart(g):
            bsel = g % 2
            cbase = pl.multiple_of(
                wid * (B_PER_TILE * S_PAD) + g * CTX_CHUNK, CTX_CHUNK)
            pltpu.sync_copy(cidx_hbm.at[pl.ds(cbase, CTX_CHUNK)], cidx[bsel])
            return pltpu.async_copy(ctab_hbm.at[cidx[bsel]], cbuf[bsel],
                                    sem_g[bsel])

        cgathers = [cstart(0)]
        pwrites = []
        for g in range(N_B_GROUPS):
            psel = g % 2
            bsel = g % 2
            cgathers[g].wait()
            if g + 1 < N_B_GROUPS:
                cgathers.append(cstart(g + 1))
            if g >= 2:
                pwrites[g - 2].wait()
            zero = jnp.zeros((16,), jnp.float32)
            for bi in range(B_GROUP):
                def body(s, acc, _bi=bi, _bsel=bsel):
                    j = _bi * S_PAD + s
                    return tuple(
                        acc[c] + cbuf[_bsel][j, pl.ds(c * 16, 16)]
                        for c in range(4))

                acc = lax.fori_loop(0, S, body, (zero, zero, zero, zero))
                for c in range(4):
                    pool[psel][bi, pl.ds(c * 16, 16)] = acc[c]
            obase = pl.multiple_of(wid * B_PER_TILE + g * B_GROUP, B_GROUP)
            pwrites.append(pltpu.async_copy(
                pool[psel], pooled_out.at[pl.ds(obase, B_GROUP)],
                sem_p[psel]))
        pwrites[N_B_GROUPS - 2].wait()
        pwrites[N_B_GROUPS - 1].wait()

    return k(ctab, cidx_flat)


def _tc_score(item2, cidx, pooled, row0, W1, b1, W2, b2):
    BB = 256

    def body(item_ref, cidx_ref, pool_ref, row0_ref, W1_ref, b1_ref,
             W2_ref, b2_ref, out_ref):
        nz = jnp.sum((cidx_ref[...] == 0).astype(jnp.float32), axis=1,
                     keepdims=True)
        pooled_c = pool_ref[...] - nz * row0_ref[...]
        up = jnp.tanh(
            jnp.dot(pooled_c, W1_ref[...],
                    preferred_element_type=jnp.float32) + b1_ref[...])
        ctx = (jnp.dot(up, W2_ref[...], preferred_element_type=jnp.float32)
               + b2_ref[...])
        item3 = item_ref[...].reshape(BB, S, E)
        diff = item3 - ctx[:, None, :]
        d2 = jnp.sum(diff * diff, axis=-1)
        out_ref[...] = 1.0 - jnp.tanh(jnp.sqrt(d2))

    return pl.pallas_call(
        body,
        grid=(B // BB,),
        in_specs=[
            pl.BlockSpec((BB * S, E), lambda i: (i, 0)),
            pl.BlockSpec((BB, S), lambda i: (i, 0)),
            pl.BlockSpec((BB, E), lambda i: (i, 0)),
            pl.BlockSpec((1, E), lambda i: (0, 0)),
            pl.BlockSpec((E, 2 * E), lambda i: (0, 0)),
            pl.BlockSpec((1, 2 * E), lambda i: (0, 0)),
            pl.BlockSpec((2 * E, E), lambda i: (0, 0)),
            pl.BlockSpec((1, E), lambda i: (0, 0)),
        ],
        out_specs=pl.BlockSpec((BB, S), lambda i: (i, 0)),
        out_shape=jax.ShapeDtypeStruct((B, S), jnp.float32),
    )(item2, cidx, pooled, row0, W1, b1, W2, b2)


def kernel(item_indices, context_indices, item_table, ctx_table, W1, b1, W2, b2):
    ii = item_indices.astype(jnp.int32)
    ci = context_indices.astype(jnp.int32)
    spread = (jnp.arange(B * (S_PAD - S), dtype=jnp.int32) * 7919) % NUM_ITEMS
    cidx_pad = jnp.concatenate([ci, spread.reshape(B, S_PAD - S)], axis=1)
    item_embeds = _sc_item_gather(item_table, ii.reshape(-1))
    pooled = _sc_ctx_pool(ctx_table, cidx_pad.reshape(-1))
    row0 = lax.slice(ctx_table, (0, 0), (1, E))
    return _tc_score(item_embeds, ci, pooled, row0,
                     W1, b1.reshape(1, -1), W2, b2.reshape(1, -1))


# speedup vs baseline: 1.1164x; 1.0635x over previous
import functools

import jax
import jax.numpy as jnp
from jax import lax
from jax.experimental import pallas as pl
from jax.experimental.pallas import tpu as pltpu
from jax.experimental.pallas import tpu_sc as plsc

NUM_ITEMS = 1000000
B = 4096
S = 50
S_PAD = 56
E = 64
NW = 32

ITEMS_PER_TILE = (B * S) // NW
ITEM_CHUNK = 320
N_ITEM_CHUNKS = ITEMS_PER_TILE // ITEM_CHUNK
B_PER_TILE = B // NW
B_GROUP = 8
N_B_GROUPS = B_PER_TILE // B_GROUP
CTX_CHUNK = B_GROUP * S_PAD

_MESH = plsc.VectorSubcoreMesh(core_axis_name="c", subcore_axis_name="s")
_SC_PARAMS = pltpu.CompilerParams(use_tc_tiling_on_sc=False)


def _sc_item_gather(itab, iidx_flat):
    @functools.partial(
        pl.kernel,
        mesh=_MESH,
        out_type=jax.ShapeDtypeStruct((B * S, 2 * E), jnp.float32),
        scratch_types=[
            pltpu.VMEM((ITEM_CHUNK,), jnp.int32),
            pltpu.VMEM((ITEM_CHUNK,), jnp.int32),
            pltpu.VMEM((ITEM_CHUNK, E), jnp.float32),
            pltpu.VMEM((ITEM_CHUNK, E), jnp.float32),
            pltpu.SemaphoreType.DMA,
            pltpu.SemaphoreType.DMA,
            pltpu.SemaphoreType.DMA,
            pltpu.SemaphoreType.DMA,
        ],
        compiler_params=_SC_PARAMS,
    )
    def k(itab_hbm, iidx_hbm, item_out,
          iidx0, iidx1, ibuf0, ibuf1, sem_g0, sem_g1, sem_w0, sem_w1):
        wid = lax.axis_index("s") * 2 + lax.axis_index("c")
        iidx = (iidx0, iidx1)
        ibuf = (ibuf0, ibuf1)
        sem_g = (sem_g0, sem_g1)
        sem_w = (sem_w0, sem_w1)

        def istart(kk):
            bsel = kk % 2
            base = pl.multiple_of(wid * ITEMS_PER_TILE + kk * ITEM_CHUNK,
                                  ITEM_CHUNK)
            pltpu.sync_copy(iidx_hbm.at[pl.ds(base, ITEM_CHUNK)], iidx[bsel])
            return pltpu.async_copy(itab_hbm.at[iidx[bsel]], ibuf[bsel],
                                    sem_g[bsel])

        def iwrite(kk):
            bsel = kk % 2
            base = pl.multiple_of(wid * ITEMS_PER_TILE + kk * ITEM_CHUNK,
                                  ITEM_CHUNK)
            return pltpu.async_copy(
                ibuf[bsel],
                item_out.at[pl.ds(base, ITEM_CHUNK), pl.ds(0, E)],
                sem_w[bsel])

        gathers = [istart(0)]
        writes = []
        for kk in range(1, N_ITEM_CHUNKS):
            if kk >= 2:
                writes[kk - 2].wait()
            gathers.append(istart(kk))
            gathers[kk - 1].wait()
            writes.append(iwrite(kk - 1))
        gathers[N_ITEM_CHUNKS - 1].wait()
        writes.append(iwrite(N_ITEM_CHUNKS - 1))
        writes[N_ITEM_CHUNKS - 2].wait()
        writes[N_ITEM_CHUNKS - 1].wait()

    return k(itab, iidx_flat)


def _sc_ctx_pool(ctab, cidx_flat):
    @functools.partial(
        pl.kernel,
        mesh=_MESH,
        out_type=jax.ShapeDtypeStruct((B, E), jnp.float32),
        scratch_types=[
            pltpu.VMEM((CTX_CHUNK,), jnp.int32),
            pltpu.VMEM((CTX_CHUNK,), jnp.int32),
            pltpu.VMEM((CTX_CHUNK, E), jnp.float32),
            pltpu.VMEM((CTX_CHUNK, E), jnp.float32),
            pltpu.VMEM((B_GROUP, E), jnp.float32),
            pltpu.VMEM((B_GROUP, E), jnp.float32),
            pltpu.SemaphoreType.DMA,
            pltpu.SemaphoreType.DMA,
            pltpu.SemaphoreType.DMA,
            pltpu.SemaphoreType.DMA,
        ],
        compiler_params=_SC_PARAMS,
    )
    def k(ctab_hbm, cidx_hbm, pooled_out,
          cidx0, cidx1, cbuf0, cbuf1, pool0, pool1,
          sem_g0, sem_g1, sem_p0, sem_p1):
        wid = lax.axis_index("s") * 2 + lax.axis_index("c")
        cidx = (cidx0, cidx1)
        cbuf = (cbuf0, cbuf1)
        pool = (pool0, pool1)
        sem_g = (sem_g0, sem_g1)
        sem_p = (sem_p0, sem_p1)

        def cstart(g):
            bsel = g % 2
            cbase = pl.multiple_of(
                wid * (B_PER_TILE * S_PAD) + g * CTX_CHUNK, CTX_CHUNK)
            pltpu.sync_copy(cidx_hbm.at[pl.ds(cbase, CTX_CHUNK)], cidx[bsel])
            return pltpu.async_copy(ctab_hbm.at[cidx[bsel]], cbuf[bsel],
                                    sem_g[bsel])

        cgathers = [cstart(0)]
        pwrites = []
        for g in range(N_B_GROUPS):
            psel = g % 2
            bsel = g % 2
            cgathers[g].wait()
            if g + 1 < N_B_GROUPS:
                cgathers.append(cstart(g + 1))
            if g >= 2:
                pwrites[g - 2].wait()
            zero = jnp.zeros((16,), jnp.float32)
            for bi in range(B_GROUP):
                def body(s, acc, _bi=bi, _bsel=bsel):
                    j = _bi * S_PAD + s
                    return tuple(
                        acc[c] + cbuf[_bsel][j, pl.ds(c * 16, 16)]
                        for c in range(4))

                acc = lax.fori_loop(0, S, body, (zero, zero, zero, zero))
                for c in range(4):
                    pool[psel][bi, pl.ds(c * 16, 16)] = acc[c]
            obase = pl.multiple_of(wid * B_PER_TILE + g * B_GROUP, B_GROUP)
            pwrites.append(pltpu.async_copy(
                pool[psel], pooled_out.at[pl.ds(obase, B_GROUP)],
                sem_p[psel]))
        pwrites[N_B_GROUPS - 2].wait()
        pwrites[N_B_GROUPS - 1].wait()

    return k(ctab, cidx_flat)


def _tc_score(item2, cidx, pooled, row0, W1, b1, W2, b2):
    BB = 256

    def body(item_ref, cidx_ref, pool_ref, row0_ref, W1_ref, b1_ref,
             W2_ref, b2_ref, out_ref):
        nz = jnp.sum((cidx_ref[...] == 0).astype(jnp.float32), axis=1,
                     keepdims=True)
        pooled_c = pool_ref[...] - nz * row0_ref[...]
        up = jnp.tanh(
            jnp.dot(pooled_c, W1_ref[...],
                    preferred_element_type=jnp.float32) + b1_ref[...])
        ctx = (jnp.dot(up, W2_ref[...], preferred_element_type=jnp.float32)
               + b2_ref[...])
        item3 = item_ref[...][:, :E].reshape(BB, S, E)
        diff = item3 - ctx[:, None, :]
        d2 = jnp.sum(diff * diff, axis=-1)
        out_ref[...] = 1.0 - jnp.tanh(jnp.sqrt(d2))

    return pl.pallas_call(
        body,
        grid=(B // BB,),
        in_specs=[
            pl.BlockSpec((BB * S, 2 * E), lambda i: (i, 0)),
            pl.BlockSpec((BB, S), lambda i: (i, 0)),
            pl.BlockSpec((BB, E), lambda i: (i, 0)),
            pl.BlockSpec((1, E), lambda i: (0, 0)),
            pl.BlockSpec((E, 2 * E), lambda i: (0, 0)),
            pl.BlockSpec((1, 2 * E), lambda i: (0, 0)),
            pl.BlockSpec((2 * E, E), lambda i: (0, 0)),
            pl.BlockSpec((1, E), lambda i: (0, 0)),
        ],
        out_specs=pl.BlockSpec((BB, S), lambda i: (i, 0)),
        out_shape=jax.ShapeDtypeStruct((B, S), jnp.float32),
    )(item2, cidx, pooled, row0, W1, b1, W2, b2)


def kernel(item_indices, context_indices, item_table, ctx_table, W1, b1, W2, b2):
    ii = item_indices.astype(jnp.int32)
    ci = context_indices.astype(jnp.int32)
    spread = (jnp.arange(B * (S_PAD - S), dtype=jnp.int32) * 7919) % NUM_ITEMS
    cidx_pad = jnp.concatenate([ci, spread.reshape(B, S_PAD - S)], axis=1)
    item_embeds = _sc_item_gather(item_table, ii.reshape(-1))
    pooled = _sc_ctx_pool(ctx_table, cidx_pad.reshape(-1))
    row0 = lax.slice(ctx_table, (0, 0), (1, E))
    return _tc_score(item_embeds, ci, pooled, row0,
                     W1, b1.reshape(1, -1), W2, b2.reshape(1, -1))
